# baseline (speedup 1.0000x reference)
import functools

import jax
import jax.numpy as jnp
from jax import lax
from jax.experimental import pallas as pl
from jax.experimental.pallas import tpu as pltpu
from jax.experimental.pallas import tpu_sc as plsc

VOCAB = 1000000
EMB = 32
PADW = 128
N_LABELS = 100
B, L = 4096, 200

NC, NS = 2, 16
NW = NC * NS
ROWS_PER_W = B // NW
LH = 104
NPAD = 2 * LH - L
HALVES_PER_W = 2 * ROWS_PER_W

_C = 768
_NCHUNK = 999936 // _C
_TAIL0 = _NCHUNK * _C
_TAILV = VOCAB - _TAIL0
_NPAIR = (_NCHUNK // NW + 2) // 2


def _conv_chunk(cbuf, obuf, c):
    ilo = lax.iota(jnp.int32, 16) * c
    ihi = ilo + 16 * c

    def step(v8, _):
        vb = jnp.full((16,), 0, jnp.int32) + v8 * 8
        for k in range(8):
            lo = plsc.load_gather(cbuf, [ilo + (vb + k)])
            hi = plsc.load_gather(cbuf, [ihi + (vb + k)])
            off = v8 * 256 + 32 * k
            obuf[pl.ds(off, 16)] = lo
            obuf[pl.ds(off + 16, 16)] = hi
        return 0

    lax.fori_loop(0, c // 8, step, 0)


_CW = _C * EMB


def _detile_body(tt_hbm, tail_hbm, out_hbm,
                 ca, cb, oa, ob, tc_v, to_v, sia, sib, soa, sob):
    wid = lax.axis_index("s") * NC + lax.axis_index("c")

    def fire_in(g, cbuf, sem):
        for e in range(EMB):
            pltpu.async_copy(tt_hbm.at[e, pl.ds(g * _C, _C)],
                             cbuf.at[pl.ds(e * _C, _C)], sem)

    def drain_in(cbuf, sem):
        pltpu.make_async_copy(out_hbm.at[pl.ds(0, _CW)], cbuf, sem).wait()

    def fire_out(g, obuf, sem):
        pltpu.async_copy(obuf, out_hbm.at[pl.ds(g * _CW, _CW)], sem)

    def drain_out(obuf, sem):
        pltpu.make_async_copy(obuf, out_hbm.at[pl.ds(0, _CW)], sem).wait()

    fire_in(wid, ca, sia)

    def pair(p, _):
        g_a = wid + NW * 2 * p
        g_b = wid + NW * (2 * p + 1)
        g_a2 = wid + NW * (2 * p + 2)

        @pl.when(g_b < _NCHUNK)
        def _():
            fire_in(g_b, cb, sib)

        @pl.when(g_a < _NCHUNK)
        def _():
            drain_in(ca, sia)

            @pl.when(p > 0)
            def _():
                drain_out(oa, soa)

            _conv_chunk(ca, oa, _C)
            fire_out(g_a, oa, soa)

        @pl.when(g_a2 < _NCHUNK)
        def _():
            fire_in(g_a2, ca, sia)

        @pl.when(g_b < _NCHUNK)
        def _():
            drain_in(cb, sib)

            @pl.when(p > 0)
            def _():
                drain_out(ob, sob)

            _conv_chunk(cb, ob, _C)
            fire_out(g_b, ob, sob)

        return 0

    lax.fori_loop(0, _NPAIR, pair, 0)

    drain_out(oa, soa)
    drain_out(ob, sob)

    @pl.when(wid == NW - 1)
    def _():
        pltpu.sync_copy(tail_hbm, tc_v)
        _conv_chunk(tc_v, to_v, _TAILV)
        pltpu.sync_copy(to_v, out_hbm.at[pl.ds(_TAIL0 * EMB, _TAILV * EMB)])


_detile = functools.partial(
    pl.kernel,
    mesh=plsc.VectorSubcoreMesh(core_axis_name="c", subcore_axis_name="s"),
    out_type=jax.ShapeDtypeStruct((VOCAB * EMB,), jnp.float32),
    scratch_types=[
        pltpu.VMEM((_CW,), jnp.float32),
        pltpu.VMEM((_CW,), jnp.float32),
        pltpu.VMEM((_CW,), jnp.float32),
        pltpu.VMEM((_CW,), jnp.float32),
        pltpu.VMEM((_TAILV * EMB,), jnp.float32),
        pltpu.VMEM((_TAILV * EMB,), jnp.float32),
        pltpu.SemaphoreType.DMA,
        pltpu.SemaphoreType.DMA,
        pltpu.SemaphoreType.DMA,
        pltpu.SemaphoreType.DMA,
    ],
    compiler_params=pltpu.CompilerParams(use_tc_tiling_on_sc=True,
                                         needs_layout_passes=False),
)(_detile_body)


def _ldf32(buf, l, half):
    return buf[l, pl.ds(16 * half, 16)]


def _accum_row(buf0, buf1, sums_v, r):
    z = jnp.zeros((16,), jnp.float32)
    a0 = a1 = b0 = b1 = c0 = c1 = d0 = d1 = z
    for l in range(0, LH, 2):
        a0 = a0 + _ldf32(buf0, l, 0)
        a1 = a1 + _ldf32(buf0, l, 1)
        b0 = b0 + _ldf32(buf0, l + 1, 0)
        b1 = b1 + _ldf32(buf0, l + 1, 1)
        if l + 1 < LH - NPAD:
            c0 = c0 + _ldf32(buf1, l, 0)
            c1 = c1 + _ldf32(buf1, l, 1)
            d0 = d0 + _ldf32(buf1, l + 1, 0)
            d1 = d1 + _ldf32(buf1, l + 1, 1)
    sums_v[r, pl.ds(0, 16)] = (a0 + b0) + (c0 + d0)
    sums_v[r, pl.ds(16, 16)] = (a1 + b1) + (c1 + d1)


def _pool_body(table_hbm, idx_hbm, out_hbm, idx_v,
               a0_v, a1_v, b0_v, b1_v, sums_v, sem_a, sem_b):
    wid = lax.axis_index("s") * NC + lax.axis_index("c")
    hbase = wid * HALVES_PER_W
    rbase = wid * ROWS_PER_W
    pltpu.sync_copy(idx_hbm.at[pl.ds(hbase, HALVES_PER_W), :], idx_v)

    def fire(rr, b0, b1, sem):
        pltpu.async_copy(table_hbm.at[idx_v.at[rr]], b0, sem)
        pltpu.async_copy(table_hbm.at[idx_v.at[rr + 1]], b1, sem)

    def drain(b0, b1, sem):
        pltpu.make_async_copy(table_hbm.at[pl.ds(0, LH)], b0, sem).wait()
        pltpu.make_async_copy(table_hbm.at[pl.ds(0, LH)], b1, sem).wait()

    fire(0, a0_v, a1_v, sem_a)

    def pair(p, _):
        fire(4 * p + 2, b0_v, b1_v, sem_b)
        drain(a0_v, a1_v, sem_a)
        _accum_row(a0_v, a1_v, sums_v, 2 * p)

        @pl.when(p < ROWS_PER_W // 2 - 1)
        def _():
            fire(4 * p + 4, a0_v, a1_v, sem_a)

        drain(b0_v, b1_v, sem_b)
        _accum_row(b0_v, b1_v, sums_v, 2 * p + 1)
        return 0

    lax.fori_loop(0, ROWS_PER_W // 2, pair, 0)
    pltpu.sync_copy(sums_v, out_hbm.at[pl.ds(rbase, ROWS_PER_W), :])


_pool = functools.partial(
    pl.kernel,
    mesh=plsc.VectorSubcoreMesh(core_axis_name="c", subcore_axis_name="s"),
    out_type=jax.ShapeDtypeStruct((B, EMB), jnp.float32),
    scratch_types=[
        pltpu.VMEM((HALVES_PER_W, LH), jnp.int32),
        pltpu.VMEM((LH, EMB), jnp.float32),
        pltpu.VMEM((LH, EMB), jnp.float32),
        pltpu.VMEM((LH, EMB), jnp.float32),
        pltpu.VMEM((LH, EMB), jnp.float32),
        pltpu.VMEM((ROWS_PER_W, EMB), jnp.float32),
        pltpu.SemaphoreType.DMA,
        pltpu.SemaphoreType.DMA,
    ],
    compiler_params=pltpu.CompilerParams(use_tc_tiling_on_sc=False),
)(_pool_body)


def _head_body(x_ref, sums_ref, w_ref, b_ref, out_ref):
    mask = (x_ref[...] != 0).astype(jnp.float32)
    cnt = jnp.maximum(jnp.sum(mask, axis=1, keepdims=True), 1.0)
    avg = sums_ref[...] / cnt
    out_ref[...] = lax.dot_general(
        avg, w_ref[...], (((1,), (1,)), ((), ())),
        preferred_element_type=jnp.float32,
    ) + b_ref[...]


_HEAD_BLK = 512


def _head(x, sums, fc_w, fc_b2):
    return pl.pallas_call(
        _head_body,
        grid=(B // _HEAD_BLK,),
        in_specs=[
            pl.BlockSpec((_HEAD_BLK, L), lambda i: (i, 0)),
            pl.BlockSpec((_HEAD_BLK, EMB), lambda i: (i, 0)),
            pl.BlockSpec((N_LABELS, EMB), lambda i: (0, 0)),
            pl.BlockSpec((1, N_LABELS), lambda i: (0, 0)),
        ],
        out_specs=pl.BlockSpec((_HEAD_BLK, N_LABELS), lambda i: (i, 0)),
        out_shape=jax.ShapeDtypeStruct((B, N_LABELS), jnp.float32),
    )(x, sums, fc_w, fc_b2)


@jax.jit
def kernel(x, emb_table, fc_w, fc_b):
    dummy = (jnp.arange(B * NPAD, dtype=jnp.int32) % VOCAB).reshape(B, NPAD)
    xp = jnp.concatenate([x, dummy], axis=1).reshape(2 * B, LH)
    tail = emb_table[_TAIL0:, :].T.reshape(_TAILV * EMB)
    tbl1 = _detile(emb_table.T, tail)
    tbl = tbl1.reshape(VOCAB, EMB)
    sums = _pool(tbl, xp)
    return _head(x, sums, fc_w, fc_b.reshape(1, N_LABELS))


# speedup vs baseline: 1.3007x; 1.3007x over previous
import functools

import jax
import jax.numpy as jnp
from jax import lax
from jax.experimental import pallas as pl
from jax.experimental.pallas import tpu as pltpu
from jax.experimental.pallas import tpu_sc as plsc

VOCAB = 1000000
EMB = 32
N_LABELS = 100
B, L = 4096, 200

NC, NS = 2, 16
NW = NC * NS
ROWS_PER_W = B // NW
LH = 104
NPAD = 2 * LH - L
HALVES_PER_W = 2 * ROWS_PER_W


def _accum_row(buf0, buf1, sums_v, r):
    z = jnp.zeros((16,), jnp.float32)
    a0 = a1 = b0 = b1 = c0 = c1 = d0 = d1 = z
    for l in range(0, LH, 2):
        a0 = a0 + buf0[l, pl.ds(0, 16)]
        a1 = a1 + buf0[l, pl.ds(16, 16)]
        b0 = b0 + buf0[l + 1, pl.ds(0, 16)]
        b1 = b1 + buf0[l + 1, pl.ds(16, 16)]
        if l + 1 < LH - NPAD:
            c0 = c0 + buf1[l, pl.ds(0, 16)]
            c1 = c1 + buf1[l, pl.ds(16, 16)]
            d0 = d0 + buf1[l + 1, pl.ds(0, 16)]
            d1 = d1 + buf1[l + 1, pl.ds(16, 16)]
    sums_v[r, pl.ds(0, 16)] = (a0 + b0) + (c0 + d0)
    sums_v[r, pl.ds(16, 16)] = (a1 + b1) + (c1 + d1)


def _pool_body(table_hbm, idx_hbm, out_hbm, idx_v,
               a0_v, a1_v, b0_v, b1_v, sums_v, sem_a, sem_b):
    wid = lax.axis_index("s") * NC + lax.axis_index("c")
    hbase = wid * HALVES_PER_W
    rbase = wid * ROWS_PER_W
    pltpu.sync_copy(idx_hbm.at[pl.ds(hbase, HALVES_PER_W), :], idx_v)

    def fire(rr, b0, b1, sem):
        pltpu.async_copy(table_hbm.at[idx_v.at[rr]], b0, sem)
        pltpu.async_copy(table_hbm.at[idx_v.at[rr + 1]], b1, sem)

    def drain(b0, b1, sem):
        pltpu.make_async_copy(table_hbm.at[pl.ds(0, LH)], b0, sem).wait()
        pltpu.make_async_copy(table_hbm.at[pl.ds(0, LH)], b1, sem).wait()

    fire(0, a0_v, a1_v, sem_a)

    def pair(p, _):
        fire(4 * p + 2, b0_v, b1_v, sem_b)
        drain(a0_v, a1_v, sem_a)
        _accum_row(a0_v, a1_v, sums_v, 2 * p)

        @pl.when(p < ROWS_PER_W // 2 - 1)
        def _():
            fire(4 * p + 4, a0_v, a1_v, sem_a)

        drain(b0_v, b1_v, sem_b)
        _accum_row(b0_v, b1_v, sums_v, 2 * p + 1)
        return 0

    lax.fori_loop(0, ROWS_PER_W // 2, pair, 0)
    pltpu.sync_copy(sums_v, out_hbm.at[pl.ds(rbase, ROWS_PER_W), :])


_pool = functools.partial(
    pl.kernel,
    mesh=plsc.VectorSubcoreMesh(core_axis_name="c", subcore_axis_name="s"),
    out_type=jax.ShapeDtypeStruct((B, EMB), jnp.float32),
    scratch_types=[
        pltpu.VMEM((HALVES_PER_W, LH), jnp.int32),
        pltpu.VMEM((LH, EMB), jnp.float32),
        pltpu.VMEM((LH, EMB), jnp.float32),
        pltpu.VMEM((LH, EMB), jnp.float32),
        pltpu.VMEM((LH, EMB), jnp.float32),
        pltpu.VMEM((ROWS_PER_W, EMB), jnp.float32),
        pltpu.SemaphoreType.DMA,
        pltpu.SemaphoreType.DMA,
    ],
    compiler_params=pltpu.CompilerParams(use_tc_tiling_on_sc=False),
)(_pool_body)


def _head_body(x_ref, sums_ref, w_ref, b_ref, out_ref):
    mask = (x_ref[...] != 0).astype(jnp.float32)
    cnt = jnp.maximum(jnp.sum(mask, axis=1, keepdims=True), 1.0)
    avg = sums_ref[...] / cnt
    out_ref[...] = lax.dot_general(
        avg, w_ref[...], (((1,), (1,)), ((), ())),
        preferred_element_type=jnp.float32,
    ) + b_ref[...]


_HEAD_BLK = 512


def _head(x, sums, fc_w, fc_b2):
    return pl.pallas_call(
        _head_body,
        grid=(B // _HEAD_BLK,),
        in_specs=[
            pl.BlockSpec((_HEAD_BLK, L), lambda i: (i, 0)),
            pl.BlockSpec((_HEAD_BLK, EMB), lambda i: (i, 0)),
            pl.BlockSpec((N_LABELS, EMB), lambda i: (0, 0)),
            pl.BlockSpec((1, N_LABELS), lambda i: (0, 0)),
        ],
        out_specs=pl.BlockSpec((_HEAD_BLK, N_LABELS), lambda i: (i, 0)),
        out_shape=jax.ShapeDtypeStruct((B, N_LABELS), jnp.float32),
    )(x, sums, fc_w, fc_b2)


@jax.jit
def kernel(x, emb_table, fc_w, fc_b):
    dummy = (jnp.arange(B * NPAD, dtype=jnp.int32) % VOCAB).reshape(B, NPAD)
    xp = jnp.concatenate([x, dummy], axis=1).reshape(2 * B, LH)
    sums = _pool(emb_table, xp)
    return _head(x, sums, fc_w, fc_b.reshape(1, N_LABELS))
